# Initial kernel scaffold; baseline (speedup 1.0000x reference)
import jax
import jax.numpy as jnp
from jax.experimental import pallas as pl


def kernel(x, edge_index, W1, b1, W2, b2):
    raise NotImplementedError("write your pallas kernel here")


# speedup vs baseline: 16.0391x; 16.0391x over previous
import functools

import jax
import jax.numpy as jnp
from jax import lax
from jax.experimental import pallas as pl
from jax.experimental.pallas import tpu as pltpu
from jax.experimental.pallas import tpu_sc as plsc

N = 10000
E = 320000
IN_DIM = 128
HID_DIM = 256
OUT_DIM = 128

NC = 2
NS = 16
NW = NC * NS
EPT = E // NW
CH = 128
NCHUNK = 80
EPAD = NCHUNK * CH - EPT
NPAD_ROWS = 112
NA = N + NPAD_ROWS
RPT = NA // NS
SW = 64
RB = 1000
GRID = N // RB

_mesh = plsc.VectorSubcoreMesh(core_axis_name="c", subcore_axis_name="s")


def _fill_const(ref, rows, cols, val):
    vec = jnp.full((16,), val, jnp.float32)

    def body(i, _):
        for k in range(cols // 16):
            ref[i, pl.ds(k * 16, 16)] = vec
        return 0

    lax.fori_loop(0, rows, body, 0)


def _zero_acc_slice(z_v, acc_sh, sid):
    nfull, rem = RPT // CH, RPT % CH
    for r0 in range(nfull):
        pltpu.sync_copy(z_v, acc_sh.at[pl.ds(sid * RPT + r0 * CH, CH)])
    if rem:
        pltpu.sync_copy(z_v.at[pl.ds(0, rem)],
                        acc_sh.at[pl.ds(sid * RPT + nfull * CH, rem)])


def _deg_body(dst_hbm, out_hbm, dst_v, ones_v, z_v, acc_sh):
    cid = lax.axis_index("c")
    sid = lax.axis_index("s")
    pltpu.sync_copy(dst_hbm.at[cid, sid], dst_v)
    _fill_const(ones_v, CH, 16, 1.0)
    _fill_const(z_v, CH, 16, 0.0)
    _zero_acc_slice(z_v, acc_sh, sid)
    plsc.subcore_barrier()

    def chunk(j, _):
        pltpu.sync_copy(ones_v, acc_sh.at[dst_v.at[j]], add=True)
        return 0

    lax.fori_loop(0, NCHUNK, chunk, 0)
    plsc.subcore_barrier()
    pltpu.sync_copy(acc_sh.at[pl.ds(sid * RPT, RPT)],
                    out_hbm.at[cid, pl.ds(sid * RPT, RPT)])


_deg_kernel = functools.partial(
    pl.kernel,
    out_type=jax.ShapeDtypeStruct((NC, NA, 16), jnp.float32),
    mesh=_mesh,
    scratch_types=[
        pltpu.VMEM((NCHUNK, CH), jnp.int32),
        pltpu.VMEM((CH, 16), jnp.float32),
        pltpu.VMEM((CH, 16), jnp.float32),
        pltpu.VMEM_SHARED((NA, 16), jnp.float32),
    ],
    compiler_params=pltpu.CompilerParams(use_tc_tiling_on_sc=False),
)(_deg_body)


def _make_agg_body(n_slabs):
    def body(*args):
        src_hbm, dst_hbm = args[0], args[1]
        h_hbms = args[2:2 + n_slabs]
        out_hbm = args[2 + n_slabs]
        src_v, dst_v, rows0, rows1, z_v, acc_sh, sem0, sem1 = args[3 + n_slabs:]
        cid = lax.axis_index("c")
        sid = lax.axis_index("s")
        pltpu.sync_copy(src_hbm.at[cid, sid], src_v)
        pltpu.sync_copy(dst_hbm.at[cid, sid], dst_v)
        _fill_const(z_v, CH, SW, 0.0)
        for s in range(n_slabs):
            h_hbm = h_hbms[s]
            _zero_acc_slice(z_v, acc_sh, sid)
            plsc.subcore_barrier()

            def chunk2(j2, _):
                j = j2 * 2
                cp0 = pltpu.async_copy(h_hbm.at[src_v.at[j]], rows0, sem0)
                cp1 = pltpu.async_copy(h_hbm.at[src_v.at[j + 1]], rows1, sem1)
                cp0.wait()
                pltpu.sync_copy(rows0, acc_sh.at[dst_v.at[j]], add=True)
                cp1.wait()
                pltpu.sync_copy(rows1, acc_sh.at[dst_v.at[j + 1]], add=True)
                return 0

            lax.fori_loop(0, NCHUNK // 2, chunk2, 0)
            plsc.subcore_barrier()
            pltpu.sync_copy(acc_sh.at[pl.ds(sid * RPT, RPT)],
                            out_hbm.at[s, cid, pl.ds(sid * RPT, RPT)])
            if s + 1 < n_slabs:
                plsc.subcore_barrier()

    return body


def _make_agg_kernel(n_slabs):
    return functools.partial(
        pl.kernel,
        out_type=jax.ShapeDtypeStruct((n_slabs, NC, NA, SW), jnp.float32),
        mesh=_mesh,
        scratch_types=[
            pltpu.VMEM((NCHUNK, CH), jnp.int32),
            pltpu.VMEM((NCHUNK, CH), jnp.int32),
            pltpu.VMEM((CH, SW), jnp.float32),
            pltpu.VMEM((CH, SW), jnp.float32),
            pltpu.VMEM((CH, SW), jnp.float32),
            pltpu.VMEM_SHARED((NA, SW), jnp.float32),
            pltpu.SemaphoreType.DMA,
            pltpu.SemaphoreType.DMA,
        ],
        compiler_params=pltpu.CompilerParams(use_tc_tiling_on_sc=False),
    )(_make_agg_body(n_slabs))


_agg1_kernel = _make_agg_kernel(HID_DIM // SW)
_agg2_kernel = _make_agg_kernel(OUT_DIM // SW)


def _mm_body(x_ref, w_ref, o_ref):
    o_ref[...] = jnp.dot(x_ref[...], w_ref[...],
                         preferred_element_type=jnp.float32)


def _stage_b_body(degp_ref, h1_ref, dinv_ref, *hp_refs):
    deg = jnp.sum(degp_ref[...], axis=(0, 2)) * 0.0625 + 1.0
    dinv = lax.rsqrt(jnp.maximum(deg, 1.0))
    dinv_ref[...] = dinv[:, None]
    hp = dinv[:, None] * h1_ref[...]
    for k, r in enumerate(hp_refs):
        r[...] = hp[:, k * SW:(k + 1) * SW]


def _stage_c_body(agg_ref, h0_ref, h1_ref, h2_ref, h3_ref, dinv_ref, b1_ref,
                  w2_ref, o0_ref, o1_ref):
    dinv = dinv_ref[...]
    hs = [h0_ref, h1_ref, h2_ref, h3_ref]
    cols = []
    for k in range(HID_DIM // SW):
        a = agg_ref[k, 0] + agg_ref[k, 1]
        o1 = dinv * (a + hs[k][...]) + b1_ref[0, k * SW:(k + 1) * SW][None, :]
        cols.append(jnp.maximum(o1, 0.0))
    h = jnp.concatenate(cols, axis=-1)
    h2p = dinv * jnp.dot(h, w2_ref[...], preferred_element_type=jnp.float32)
    o0_ref[...] = h2p[:, :SW]
    o1_ref[...] = h2p[:, SW:]


def _stage_d_body(agg_ref, h0_ref, h1_ref, dinv_ref, b2_ref, o_ref):
    a0 = agg_ref[0, 0] + agg_ref[0, 1] + h0_ref[...]
    a1 = agg_ref[1, 0] + agg_ref[1, 1] + h1_ref[...]
    o_ref[...] = dinv_ref[...] * jnp.concatenate([a0, a1], axis=-1) \
        + b2_ref[...]


def _prep_edges(edge_index):
    ei = edge_index.astype(jnp.int32)
    src = ei[0].reshape(NC, NS, EPT)
    dst = ei[1].reshape(NC, NS, EPT)
    wid = (jnp.arange(NC)[:, None, None] * NS
           + jnp.arange(NS)[None, :, None]).astype(jnp.int32)
    pr = jnp.arange(EPAD, dtype=jnp.int32)[None, None, :]
    pad_src = (wid * 313 + pr * 97) % N
    pad_dst = jnp.broadcast_to(N + (pr % NPAD_ROWS), (NC, NS, EPAD))
    srcp = jnp.concatenate([src, jnp.broadcast_to(pad_src, (NC, NS, EPAD))],
                           axis=2).reshape(NC, NS, NCHUNK, CH)
    dstp = jnp.concatenate([dst, pad_dst], axis=2).reshape(NC, NS, NCHUNK, CH)
    return srcp, dstp


def kernel(x, edge_index, W1, b1, W2, b2):
    f32 = jnp.float32
    srcp, dstp = _prep_edges(edge_index)

    deg16 = _deg_kernel(dstp)

    h1 = pl.pallas_call(
        _mm_body,
        grid=(GRID,),
        in_specs=[pl.BlockSpec((RB, IN_DIM), lambda i: (i, 0)),
                  pl.BlockSpec((IN_DIM, HID_DIM), lambda i: (0, 0))],
        out_specs=pl.BlockSpec((RB, HID_DIM), lambda i: (i, 0)),
        out_shape=jax.ShapeDtypeStruct((N, HID_DIM), f32),
    )(x, W1)

    nsl1 = HID_DIM // SW
    bouts = pl.pallas_call(
        _stage_b_body,
        grid=(GRID,),
        in_specs=[pl.BlockSpec((NC, RB, 16), lambda i: (0, i, 0)),
                  pl.BlockSpec((RB, HID_DIM), lambda i: (i, 0))],
        out_specs=[pl.BlockSpec((RB, 1), lambda i: (i, 0))]
        + [pl.BlockSpec((RB, SW), lambda i: (i, 0))] * nsl1,
        out_shape=[jax.ShapeDtypeStruct((N, 1), f32)]
        + [jax.ShapeDtypeStruct((N, SW), f32)] * nsl1,
    )(deg16, h1)
    dinv, h1ps = bouts[0], bouts[1:]

    agg1 = _agg1_kernel(srcp, dstp, *h1ps)

    h2p0, h2p1 = pl.pallas_call(
        _stage_c_body,
        grid=(GRID,),
        in_specs=[pl.BlockSpec((nsl1, NC, RB, SW), lambda i: (0, 0, i, 0))]
        + [pl.BlockSpec((RB, SW), lambda i: (i, 0))] * nsl1
        + [pl.BlockSpec((RB, 1), lambda i: (i, 0)),
           pl.BlockSpec((1, HID_DIM), lambda i: (0, 0)),
           pl.BlockSpec((HID_DIM, OUT_DIM), lambda i: (0, 0))],
        out_specs=[pl.BlockSpec((RB, SW), lambda i: (i, 0))] * 2,
        out_shape=[jax.ShapeDtypeStruct((N, SW), f32)] * 2,
    )(agg1, *h1ps, dinv, b1.reshape(1, HID_DIM), W2)

    agg2 = _agg2_kernel(srcp, dstp, h2p0, h2p1)

    out = pl.pallas_call(
        _stage_d_body,
        grid=(GRID,),
        in_specs=[pl.BlockSpec((2, NC, RB, SW), lambda i: (0, 0, i, 0)),
                  pl.BlockSpec((RB, SW), lambda i: (i, 0)),
                  pl.BlockSpec((RB, SW), lambda i: (i, 0)),
                  pl.BlockSpec((RB, 1), lambda i: (i, 0)),
                  pl.BlockSpec((1, OUT_DIM), lambda i: (0, 0))],
        out_specs=pl.BlockSpec((RB, OUT_DIM), lambda i: (i, 0)),
        out_shape=jax.ShapeDtypeStruct((N, OUT_DIM), f32),
    )(agg2, h2p0, h2p1, dinv, b2.reshape(1, OUT_DIM))
    return out


# speedup vs baseline: 23.7183x; 1.4788x over previous
import functools

import jax
import jax.numpy as jnp
from jax import lax
from jax.experimental import pallas as pl
from jax.experimental.pallas import tpu as pltpu
from jax.experimental.pallas import tpu_sc as plsc

N = 10000
E = 320000
IN_DIM = 128
HID_DIM = 256
OUT_DIM = 128

NC = 2
NS = 16
NW = NC * NS
EPT = E // NW
CH = 128
NCHUNK = 80
EPAD = NCHUNK * CH - EPT
NPAD_ROWS = 112
NA = N + NPAD_ROWS
RPT = NA // NS
SW = 64
RB = 1000
GRID = N // RB

_mesh = plsc.VectorSubcoreMesh(core_axis_name="c", subcore_axis_name="s")


def _fill_const(ref, rows, cols, val):
    vec = jnp.full((16,), val, jnp.float32)

    def body(i, _):
        for k in range(cols // 16):
            ref[i, pl.ds(k * 16, 16)] = vec
        return 0

    lax.fori_loop(0, rows, body, 0)


def _zero_acc_slice(z_v, acc_sh, sid):
    nfull, rem = RPT // CH, RPT % CH
    for r0 in range(nfull):
        pltpu.sync_copy(z_v, acc_sh.at[pl.ds(sid * RPT + r0 * CH, CH)])
    if rem:
        pltpu.sync_copy(z_v.at[pl.ds(0, rem)],
                        acc_sh.at[pl.ds(sid * RPT + nfull * CH, rem)])


def _deg_body(dst_hbm, out_hbm, dst_v, ones_v, z_v, acc_sh):
    cid = lax.axis_index("c")
    sid = lax.axis_index("s")
    pltpu.sync_copy(dst_hbm.at[cid, sid], dst_v)
    _fill_const(ones_v, CH, 16, 1.0)
    _fill_const(z_v, CH, 16, 0.0)
    _zero_acc_slice(z_v, acc_sh, sid)
    plsc.subcore_barrier()

    def chunk(j, _):
        pltpu.sync_copy(ones_v, acc_sh.at[dst_v.at[j]], add=True)
        return 0

    lax.fori_loop(0, NCHUNK, chunk, 0)
    plsc.subcore_barrier()
    pltpu.sync_copy(acc_sh.at[pl.ds(sid * RPT, RPT)],
                    out_hbm.at[cid, pl.ds(sid * RPT, RPT)])


_deg_kernel = functools.partial(
    pl.kernel,
    out_type=jax.ShapeDtypeStruct((NC, NA, 16), jnp.float32),
    mesh=_mesh,
    scratch_types=[
        pltpu.VMEM((NCHUNK, CH), jnp.int32),
        pltpu.VMEM((CH, 16), jnp.float32),
        pltpu.VMEM((CH, 16), jnp.float32),
        pltpu.VMEM_SHARED((NA, 16), jnp.float32),
    ],
    compiler_params=pltpu.CompilerParams(use_tc_tiling_on_sc=False),
)(_deg_body)


NBUF = 4


def _make_agg_body(n_slabs):
    def body(*args):
        src_hbm, dst_hbm = args[0], args[1]
        h_hbms = args[2:2 + n_slabs]
        out_hbm = args[2 + n_slabs]
        scr = args[3 + n_slabs:]
        src_v, dst_v = scr[0], scr[1]
        bufs = scr[2:2 + NBUF]
        z_v, acc_sh = scr[2 + NBUF], scr[3 + NBUF]
        sems = scr[4 + NBUF:4 + 2 * NBUF]
        cid = lax.axis_index("c")
        sid = lax.axis_index("s")
        pltpu.sync_copy(src_hbm.at[cid, sid], src_v)
        pltpu.sync_copy(dst_hbm.at[cid, sid], dst_v)
        _fill_const(z_v, CH, SW, 0.0)
        for s in range(n_slabs):
            h_hbm = h_hbms[s]
            _zero_acc_slice(z_v, acc_sh, sid)
            plsc.subcore_barrier()
            for b in range(NBUF):
                pltpu.async_copy(h_hbm.at[src_v.at[b]], bufs[b], sems[b])

            def round_body(r, _):
                j0 = r * NBUF
                for b in range(NBUF):
                    j = j0 + b
                    pltpu.make_async_copy(h_hbm.at[src_v.at[j]],
                                          bufs[b], sems[b]).wait()
                    pltpu.sync_copy(bufs[b], acc_sh.at[dst_v.at[j]], add=True)
                    nj = j + NBUF

                    @pl.when(nj < NCHUNK)
                    def _():
                        pltpu.async_copy(h_hbm.at[src_v.at[nj]],
                                         bufs[b], sems[b])
                return 0

            lax.fori_loop(0, NCHUNK // NBUF, round_body, 0)
            plsc.subcore_barrier()
            pltpu.sync_copy(acc_sh.at[pl.ds(sid * RPT, RPT)],
                            out_hbm.at[s, cid, pl.ds(sid * RPT, RPT)])
            if s + 1 < n_slabs:
                plsc.subcore_barrier()

    return body


def _make_agg_kernel(n_slabs):
    return functools.partial(
        pl.kernel,
        out_type=jax.ShapeDtypeStruct((n_slabs, NC, NA, SW), jnp.float32),
        mesh=_mesh,
        scratch_types=[
            pltpu.VMEM((NCHUNK, CH), jnp.int32),
            pltpu.VMEM((NCHUNK, CH), jnp.int32),
        ] + [pltpu.VMEM((CH, SW), jnp.float32) for _ in range(NBUF + 1)] + [
            pltpu.VMEM_SHARED((NA, SW), jnp.float32),
        ] + [pltpu.SemaphoreType.DMA for _ in range(NBUF)],
        compiler_params=pltpu.CompilerParams(use_tc_tiling_on_sc=False),
    )(_make_agg_body(n_slabs))


_agg1_kernel = _make_agg_kernel(HID_DIM // SW)
_agg2_kernel = _make_agg_kernel(OUT_DIM // SW)


def _stage_ab_body(degp_ref, x_ref, w1_ref, dinv_ref, *hp_refs):
    deg = jnp.sum(degp_ref[...], axis=(0, 2)) * 0.0625 + 1.0
    dinv = lax.rsqrt(jnp.maximum(deg, 1.0))
    dinv_ref[...] = dinv[:, None]
    h1 = jnp.dot(x_ref[...], w1_ref[...], preferred_element_type=jnp.float32)
    hp = dinv[:, None] * h1
    for k, r in enumerate(hp_refs):
        r[...] = hp[:, k * SW:(k + 1) * SW]


def _stage_c_body(agg_ref, h0_ref, h1_ref, h2_ref, h3_ref, dinv_ref, b1_ref,
                  w2_ref, o0_ref, o1_ref):
    dinv = dinv_ref[...]
    hs = [h0_ref, h1_ref, h2_ref, h3_ref]
    cols = []
    for k in range(HID_DIM // SW):
        a = agg_ref[k, 0] + agg_ref[k, 1]
        o1 = dinv * (a + hs[k][...]) + b1_ref[0, k * SW:(k + 1) * SW][None, :]
        cols.append(jnp.maximum(o1, 0.0))
    h = jnp.concatenate(cols, axis=-1)
    h2p = dinv * jnp.dot(h, w2_ref[...], preferred_element_type=jnp.float32)
    o0_ref[...] = h2p[:, :SW]
    o1_ref[...] = h2p[:, SW:]


def _stage_d_body(agg_ref, h0_ref, h1_ref, dinv_ref, b2_ref, o_ref):
    a0 = agg_ref[0, 0] + agg_ref[0, 1] + h0_ref[...]
    a1 = agg_ref[1, 0] + agg_ref[1, 1] + h1_ref[...]
    o_ref[...] = dinv_ref[...] * jnp.concatenate([a0, a1], axis=-1) \
        + b2_ref[...]


def _prep_edges(edge_index):
    ei = edge_index.astype(jnp.int32)
    src = ei[0].reshape(NC, NS, EPT)
    dst = ei[1].reshape(NC, NS, EPT)
    wid = (jnp.arange(NC)[:, None, None] * NS
           + jnp.arange(NS)[None, :, None]).astype(jnp.int32)
    pr = jnp.arange(EPAD, dtype=jnp.int32)[None, None, :]
    pad_src = (wid * 313 + pr * 97) % N
    pad_dst = jnp.broadcast_to(N + (pr % NPAD_ROWS), (NC, NS, EPAD))
    srcp = jnp.concatenate([src, jnp.broadcast_to(pad_src, (NC, NS, EPAD))],
                           axis=2).reshape(NC, NS, NCHUNK, CH)
    dstp = jnp.concatenate([dst, pad_dst], axis=2).reshape(NC, NS, NCHUNK, CH)
    return srcp, dstp


def kernel(x, edge_index, W1, b1, W2, b2):
    f32 = jnp.float32
    srcp, dstp = _prep_edges(edge_index)

    deg16 = _deg_kernel(dstp)

    nsl1 = HID_DIM // SW
    bouts = pl.pallas_call(
        _stage_ab_body,
        grid=(GRID,),
        in_specs=[pl.BlockSpec((NC, RB, 16), lambda i: (0, i, 0)),
                  pl.BlockSpec((RB, IN_DIM), lambda i: (i, 0)),
                  pl.BlockSpec((IN_DIM, HID_DIM), lambda i: (0, 0))],
        out_specs=[pl.BlockSpec((RB, 1), lambda i: (i, 0))]
        + [pl.BlockSpec((RB, SW), lambda i: (i, 0))] * nsl1,
        out_shape=[jax.ShapeDtypeStruct((N, 1), f32)]
        + [jax.ShapeDtypeStruct((N, SW), f32)] * nsl1,
    )(deg16, x, W1)
    dinv, h1ps = bouts[0], bouts[1:]

    agg1 = _agg1_kernel(srcp, dstp, *h1ps)

    h2p0, h2p1 = pl.pallas_call(
        _stage_c_body,
        grid=(GRID,),
        in_specs=[pl.BlockSpec((nsl1, NC, RB, SW), lambda i: (0, 0, i, 0))]
        + [pl.BlockSpec((RB, SW), lambda i: (i, 0))] * nsl1
        + [pl.BlockSpec((RB, 1), lambda i: (i, 0)),
           pl.BlockSpec((1, HID_DIM), lambda i: (0, 0)),
           pl.BlockSpec((HID_DIM, OUT_DIM), lambda i: (0, 0))],
        out_specs=[pl.BlockSpec((RB, SW), lambda i: (i, 0))] * 2,
        out_shape=[jax.ShapeDtypeStruct((N, SW), f32)] * 2,
    )(agg1, *h1ps, dinv, b1.reshape(1, HID_DIM), W2)

    agg2 = _agg2_kernel(srcp, dstp, h2p0, h2p1)

    out = pl.pallas_call(
        _stage_d_body,
        grid=(GRID,),
        in_specs=[pl.BlockSpec((2, NC, RB, SW), lambda i: (0, 0, i, 0)),
                  pl.BlockSpec((RB, SW), lambda i: (i, 0)),
                  pl.BlockSpec((RB, SW), lambda i: (i, 0)),
                  pl.BlockSpec((RB, 1), lambda i: (i, 0)),
                  pl.BlockSpec((1, OUT_DIM), lambda i: (0, 0))],
        out_specs=pl.BlockSpec((RB, OUT_DIM), lambda i: (i, 0)),
        out_shape=jax.ShapeDtypeStruct((N, OUT_DIM), f32),
    )(agg2, h2p0, h2p1, dinv, b2.reshape(1, OUT_DIM))
    return out


# speedup vs baseline: 25.7167x; 1.0843x over previous
import functools

import jax
import jax.numpy as jnp
from jax import lax
from jax.experimental import pallas as pl
from jax.experimental.pallas import tpu as pltpu
from jax.experimental.pallas import tpu_sc as plsc

N = 10000
E = 320000
IN_DIM = 128
HID_DIM = 256
OUT_DIM = 128

NC = 2
NS = 16
NW = NC * NS
EPT = E // NW
CH = 128
NCHUNK = 80
EPAD = NCHUNK * CH - EPT
NPAD_ROWS = 112
NA = N + NPAD_ROWS
RPT = NA // NS
SW = 64
RB = 1000
GRID = N // RB

_mesh = plsc.VectorSubcoreMesh(core_axis_name="c", subcore_axis_name="s")


def _fill_const(ref, rows, cols, val):
    vec = jnp.full((16,), val, jnp.float32)

    def body(i, _):
        for k in range(cols // 16):
            ref[i, pl.ds(k * 16, 16)] = vec
        return 0

    lax.fori_loop(0, rows, body, 0)


def _zero_acc_slice(z_v, acc_sh, sid):
    nfull, rem = RPT // CH, RPT % CH
    for r0 in range(nfull):
        pltpu.sync_copy(z_v, acc_sh.at[pl.ds(sid * RPT + r0 * CH, CH)])
    if rem:
        pltpu.sync_copy(z_v.at[pl.ds(0, rem)],
                        acc_sh.at[pl.ds(sid * RPT + nfull * CH, rem)])


def _deg_body(dst_hbm, out_hbm, dst_v, ones_v, z_v, acc_sh):
    cid = lax.axis_index("c")
    sid = lax.axis_index("s")
    pltpu.sync_copy(dst_hbm.at[cid * NS + sid], dst_v)
    _fill_const(ones_v, CH, 16, 1.0)
    _fill_const(z_v, CH, 16, 0.0)
    _zero_acc_slice(z_v, acc_sh, sid)
    plsc.subcore_barrier()

    def chunk(j, _):
        pltpu.sync_copy(ones_v, acc_sh.at[dst_v.at[j]], add=True)
        return 0

    lax.fori_loop(0, NCHUNK, chunk, 0)
    plsc.subcore_barrier()
    pltpu.sync_copy(acc_sh.at[pl.ds(sid * RPT, RPT)],
                    out_hbm.at[cid, pl.ds(sid * RPT, RPT)])


_deg_kernel = functools.partial(
    pl.kernel,
    out_type=jax.ShapeDtypeStruct((NC, NA, 16), jnp.float32),
    mesh=_mesh,
    scratch_types=[
        pltpu.VMEM((NCHUNK, CH), jnp.int32),
        pltpu.VMEM((CH, 16), jnp.float32),
        pltpu.VMEM((CH, 16), jnp.float32),
        pltpu.VMEM_SHARED((NA, 16), jnp.float32),
    ],
    compiler_params=pltpu.CompilerParams(use_tc_tiling_on_sc=False),
)(_deg_body)


NBUF = 4
GPT = NW // NS


def _make_agg_body(n_slabs):
    spc = n_slabs // NC

    def body(src_hbm, dst_hbm, h_hbm, out_hbm, src_v, dst_v, *scr):
        bufs = scr[0:NBUF]
        z_v, acc_sh = scr[NBUF], scr[NBUF + 1]
        sems = scr[NBUF + 2:NBUF + 2 + NBUF]
        cid = lax.axis_index("c")
        sid = lax.axis_index("s")
        _fill_const(z_v, CH, SW, 0.0)
        for t in range(spc):
            s = cid * spc + t
            _zero_acc_slice(z_v, acc_sh, sid)
            plsc.subcore_barrier()
            for u in range(GPT):
                g = sid * GPT + u
                pltpu.sync_copy(src_hbm.at[g], src_v)
                pltpu.sync_copy(dst_hbm.at[g], dst_v)
                for b in range(NBUF):
                    pltpu.async_copy(h_hbm.at[s].at[src_v.at[b]],
                                     bufs[b], sems[b])

                def round_body(r, _):
                    j0 = r * NBUF
                    for b in range(NBUF):
                        j = j0 + b
                        pltpu.make_async_copy(h_hbm.at[s].at[src_v.at[j]],
                                              bufs[b], sems[b]).wait()
                        pltpu.sync_copy(bufs[b], acc_sh.at[dst_v.at[j]],
                                        add=True)
                        nj = j + NBUF

                        @pl.when(nj < NCHUNK)
                        def _():
                            pltpu.async_copy(h_hbm.at[s].at[src_v.at[nj]],
                                             bufs[b], sems[b])
                    return 0

                lax.fori_loop(0, NCHUNK // NBUF, round_body, 0)
            plsc.subcore_barrier()
            pltpu.sync_copy(acc_sh.at[pl.ds(sid * RPT, RPT)],
                            out_hbm.at[s, pl.ds(sid * RPT, RPT)])
            if t + 1 < spc:
                plsc.subcore_barrier()

    return body


def _make_agg_kernel(n_slabs):
    return functools.partial(
        pl.kernel,
        out_type=jax.ShapeDtypeStruct((n_slabs, NA, SW), jnp.float32),
        mesh=_mesh,
        scratch_types=[
            pltpu.VMEM((NCHUNK, CH), jnp.int32),
            pltpu.VMEM((NCHUNK, CH), jnp.int32),
        ] + [pltpu.VMEM((CH, SW), jnp.float32) for _ in range(NBUF + 1)] + [
            pltpu.VMEM_SHARED((NA, SW), jnp.float32),
        ] + [pltpu.SemaphoreType.DMA for _ in range(NBUF)],
        compiler_params=pltpu.CompilerParams(use_tc_tiling_on_sc=False),
    )(_make_agg_body(n_slabs))


_agg1_kernel = _make_agg_kernel(HID_DIM // SW)
_agg2_kernel = _make_agg_kernel(OUT_DIM // SW)


def _stage_ab_body(degp_ref, x_ref, w1_ref, dinv_ref, hp_ref):
    deg = jnp.sum(degp_ref[...], axis=(0, 2)) * 0.0625 + 1.0
    dinv = lax.rsqrt(jnp.maximum(deg, 1.0))
    dinv_ref[...] = dinv[:, None]
    h1 = jnp.dot(x_ref[...], w1_ref[...], preferred_element_type=jnp.float32)
    hp = dinv[:, None] * h1
    for k in range(HID_DIM // SW):
        hp_ref[k] = hp[:, k * SW:(k + 1) * SW]


def _stage_c_body(agg_ref, hp_ref, dinv_ref, b1_ref, w2_ref, o_ref):
    dinv = dinv_ref[...]
    cols = []
    for k in range(HID_DIM // SW):
        o1 = dinv * (agg_ref[k] + hp_ref[k]) \
            + b1_ref[0, k * SW:(k + 1) * SW][None, :]
        cols.append(jnp.maximum(o1, 0.0))
    h = jnp.concatenate(cols, axis=-1)
    h2p = dinv * jnp.dot(h, w2_ref[...], preferred_element_type=jnp.float32)
    for k in range(OUT_DIM // SW):
        o_ref[k] = h2p[:, k * SW:(k + 1) * SW]


def _stage_d_body(agg_ref, hp_ref, dinv_ref, b2_ref, o_ref):
    a0 = agg_ref[0] + hp_ref[0]
    a1 = agg_ref[1] + hp_ref[1]
    o_ref[...] = dinv_ref[...] * jnp.concatenate([a0, a1], axis=-1) \
        + b2_ref[...]


def _prep_edges(edge_index):
    ei = edge_index.astype(jnp.int32)
    src = ei[0].reshape(NW, EPT)
    dst = ei[1].reshape(NW, EPT)
    wid = jnp.arange(NW, dtype=jnp.int32)[:, None]
    pr = jnp.arange(EPAD, dtype=jnp.int32)[None, :]
    pad_src = (wid * 313 + pr * 97) % N
    pad_dst = jnp.broadcast_to(N + (pr % NPAD_ROWS), (NW, EPAD))
    srcp = jnp.concatenate([src, jnp.broadcast_to(pad_src, (NW, EPAD))],
                           axis=1).reshape(NW, NCHUNK, CH)
    dstp = jnp.concatenate([dst, pad_dst], axis=1).reshape(NW, NCHUNK, CH)
    return srcp, dstp


def kernel(x, edge_index, W1, b1, W2, b2):
    f32 = jnp.float32
    srcp, dstp = _prep_edges(edge_index)

    deg16 = _deg_kernel(dstp)

    nsl1 = HID_DIM // SW
    nsl2 = OUT_DIM // SW
    dinv, h1p = pl.pallas_call(
        _stage_ab_body,
        grid=(GRID,),
        in_specs=[pl.BlockSpec((NC, RB, 16), lambda i: (0, i, 0)),
                  pl.BlockSpec((RB, IN_DIM), lambda i: (i, 0)),
                  pl.BlockSpec((IN_DIM, HID_DIM), lambda i: (0, 0))],
        out_specs=[pl.BlockSpec((RB, 1), lambda i: (i, 0)),
                   pl.BlockSpec((nsl1, RB, SW), lambda i: (0, i, 0))],
        out_shape=[jax.ShapeDtypeStruct((N, 1), f32),
                   jax.ShapeDtypeStruct((nsl1, N, SW), f32)],
    )(deg16, x, W1)

    agg1 = _agg1_kernel(srcp, dstp, h1p)

    h2p = pl.pallas_call(
        _stage_c_body,
        grid=(GRID,),
        in_specs=[pl.BlockSpec((nsl1, RB, SW), lambda i: (0, i, 0)),
                  pl.BlockSpec((nsl1, RB, SW), lambda i: (0, i, 0)),
                  pl.BlockSpec((RB, 1), lambda i: (i, 0)),
                  pl.BlockSpec((1, HID_DIM), lambda i: (0, 0)),
                  pl.BlockSpec((HID_DIM, OUT_DIM), lambda i: (0, 0))],
        out_specs=pl.BlockSpec((nsl2, RB, SW), lambda i: (0, i, 0)),
        out_shape=jax.ShapeDtypeStruct((nsl2, N, SW), f32),
    )(agg1, h1p, dinv, b1.reshape(1, HID_DIM), W2)

    agg2 = _agg2_kernel(srcp, dstp, h2p)

    out = pl.pallas_call(
        _stage_d_body,
        grid=(GRID,),
        in_specs=[pl.BlockSpec((nsl2, RB, SW), lambda i: (0, i, 0)),
                  pl.BlockSpec((nsl2, RB, SW), lambda i: (0, i, 0)),
                  pl.BlockSpec((RB, 1), lambda i: (i, 0)),
                  pl.BlockSpec((1, OUT_DIM), lambda i: (0, 0))],
        out_specs=pl.BlockSpec((RB, OUT_DIM), lambda i: (i, 0)),
        out_shape=jax.ShapeDtypeStruct((N, OUT_DIM), f32),
    )(agg2, h2p, dinv, b2.reshape(1, OUT_DIM))
    return out


# speedup vs baseline: 34.9633x; 1.3596x over previous
import functools

import jax
import jax.numpy as jnp
from jax import lax
from jax.experimental import pallas as pl
from jax.experimental.pallas import tpu as pltpu
from jax.experimental.pallas import tpu_sc as plsc

N = 10000
E = 320000
IN_DIM = 128
HID_DIM = 256
OUT_DIM = 128

NC = 2
NS = 16
NW = NC * NS
EPT = E // NW
CH = 128
NCHUNK = 80
EPAD = NCHUNK * CH - EPT
NPAD_ROWS = 112
NA = N + NPAD_ROWS
RPT = NA // NS
SW = 64
RB = 1000
GRID = N // RB

_mesh = plsc.VectorSubcoreMesh(core_axis_name="c", subcore_axis_name="s")


def _fill_const(ref, rows, cols, val):
    vec = jnp.full((16,), val, jnp.float32)

    def body(i, _):
        for k in range(cols // 16):
            ref[i, pl.ds(k * 16, 16)] = vec
        return 0

    lax.fori_loop(0, rows, body, 0)


def _zero_acc_slice(z_v, acc_sh, sid):
    nfull, rem = RPT // CH, RPT % CH
    for r0 in range(nfull):
        pltpu.sync_copy(z_v, acc_sh.at[pl.ds(sid * RPT + r0 * CH, CH)])
    if rem:
        pltpu.sync_copy(z_v.at[pl.ds(0, rem)],
                        acc_sh.at[pl.ds(sid * RPT + nfull * CH, rem)])


def _deg_body(dst_hbm, out_hbm, dst_v, ones_v, z_v, acc_sh):
    cid = lax.axis_index("c")
    sid = lax.axis_index("s")
    pltpu.sync_copy(dst_hbm.at[cid * NS + sid], dst_v)
    _fill_const(ones_v, CH, 16, 1.0)
    _fill_const(z_v, CH, 16, 0.0)
    _zero_acc_slice(z_v, acc_sh, sid)
    plsc.subcore_barrier()

    def chunk(j, _):
        pltpu.sync_copy(ones_v, acc_sh.at[dst_v.at[j]], add=True)
        return 0

    lax.fori_loop(0, NCHUNK, chunk, 0)
    plsc.subcore_barrier()
    pltpu.sync_copy(acc_sh.at[pl.ds(sid * RPT, RPT)],
                    out_hbm.at[cid, pl.ds(sid * RPT, RPT)])


_deg_kernel = functools.partial(
    pl.kernel,
    out_type=jax.ShapeDtypeStruct((NC, NA, 16), jnp.float32),
    mesh=_mesh,
    scratch_types=[
        pltpu.VMEM((NCHUNK, CH), jnp.int32),
        pltpu.VMEM((CH, 16), jnp.float32),
        pltpu.VMEM((CH, 16), jnp.float32),
        pltpu.VMEM_SHARED((NA, 16), jnp.float32),
    ],
    compiler_params=pltpu.CompilerParams(use_tc_tiling_on_sc=False),
)(_deg_body)


NBUF = 4
GPT = NW // NS


def _make_agg_body(n_slabs):
    spc = n_slabs // NC

    def body(src_hbm, dst_hbm, h_hbm, out_hbm, src_v, dst_v, *scr):
        bufs = scr[0:NBUF]
        z_v, acc_sh = scr[NBUF], scr[NBUF + 1]
        sems = scr[NBUF + 2:NBUF + 2 + NBUF]
        cid = lax.axis_index("c")
        sid = lax.axis_index("s")
        _fill_const(z_v, CH, SW, 0.0)
        for t in range(spc):
            s = cid * spc + t
            _zero_acc_slice(z_v, acc_sh, sid)
            plsc.subcore_barrier()
            for u in range(GPT):
                g = sid * GPT + u
                pltpu.sync_copy(src_hbm.at[g], src_v)
                pltpu.sync_copy(dst_hbm.at[g], dst_v)
                for b in range(NBUF):
                    pltpu.async_copy(h_hbm.at[s].at[src_v.at[b]],
                                     bufs[b], sems[b])

                def round_body(r, _):
                    j0 = r * NBUF
                    for b in range(NBUF):
                        j = j0 + b
                        pltpu.make_async_copy(h_hbm.at[s].at[src_v.at[j]],
                                              bufs[b], sems[b]).wait()
                        pltpu.sync_copy(bufs[b], acc_sh.at[dst_v.at[j]],
                                        add=True)
                        nj = j + NBUF

                        @pl.when(nj < NCHUNK)
                        def _():
                            pltpu.async_copy(h_hbm.at[s].at[src_v.at[nj]],
                                             bufs[b], sems[b])
                    return 0

                lax.fori_loop(0, NCHUNK // NBUF, round_body, 0)
            plsc.subcore_barrier()
            pltpu.sync_copy(acc_sh.at[pl.ds(sid * RPT, RPT)],
                            out_hbm.at[s, pl.ds(sid * RPT, RPT)])
            if t + 1 < spc:
                plsc.subcore_barrier()

    return body


def _make_agg_kernel(n_slabs):
    return functools.partial(
        pl.kernel,
        out_type=jax.ShapeDtypeStruct((n_slabs, NA, SW), jnp.float32),
        mesh=_mesh,
        scratch_types=[
            pltpu.VMEM((NCHUNK, CH), jnp.int32),
            pltpu.VMEM((NCHUNK, CH), jnp.int32),
        ] + [pltpu.VMEM((CH, SW), jnp.float32) for _ in range(NBUF + 1)] + [
            pltpu.VMEM_SHARED((NA, SW), jnp.float32),
        ] + [pltpu.SemaphoreType.DMA for _ in range(NBUF)],
        compiler_params=pltpu.CompilerParams(use_tc_tiling_on_sc=False),
    )(_make_agg_body(n_slabs))


_agg_kernel = _make_agg_kernel(IN_DIM // SW)


def _stage_ab_body(degp_ref, x_ref, dinv_ref, xp_ref):
    deg = jnp.sum(degp_ref[...], axis=(0, 2)) * 0.0625 + 1.0
    dinv = lax.rsqrt(jnp.maximum(deg, 1.0))
    dinv_ref[...] = dinv[:, None]
    xp = dinv[:, None] * x_ref[...]
    for k in range(IN_DIM // SW):
        xp_ref[k] = xp[:, k * SW:(k + 1) * SW]


def _stage_c_body(agg_ref, xp_ref, dinv_ref, b1_ref, w1_ref, w2_ref, o_ref):
    dinv = dinv_ref[...]
    ax = jnp.concatenate([agg_ref[k] + xp_ref[k]
                          for k in range(IN_DIM // SW)], axis=-1)
    o1 = dinv * jnp.dot(ax, w1_ref[...],
                        preferred_element_type=jnp.float32) + b1_ref[...]
    h = jnp.maximum(o1, 0.0)
    h2p = dinv * jnp.dot(h, w2_ref[...], preferred_element_type=jnp.float32)
    for k in range(OUT_DIM // SW):
        o_ref[k] = h2p[:, k * SW:(k + 1) * SW]


def _stage_d_body(agg_ref, hp_ref, dinv_ref, b2_ref, o_ref):
    a0 = agg_ref[0] + hp_ref[0]
    a1 = agg_ref[1] + hp_ref[1]
    o_ref[...] = dinv_ref[...] * jnp.concatenate([a0, a1], axis=-1) \
        + b2_ref[...]


def _prep_edges(edge_index):
    ei = edge_index.astype(jnp.int32)
    src = ei[0].reshape(NW, EPT)
    dst = ei[1].reshape(NW, EPT)
    wid = jnp.arange(NW, dtype=jnp.int32)[:, None]
    pr = jnp.arange(EPAD, dtype=jnp.int32)[None, :]
    pad_src = (wid * 313 + pr * 97) % N
    pad_dst = jnp.broadcast_to(N + (pr % NPAD_ROWS), (NW, EPAD))
    srcp = jnp.concatenate([src, jnp.broadcast_to(pad_src, (NW, EPAD))],
                           axis=1).reshape(NW, NCHUNK, CH)
    dstp = jnp.concatenate([dst, pad_dst], axis=1).reshape(NW, NCHUNK, CH)
    return srcp, dstp


def kernel(x, edge_index, W1, b1, W2, b2):
    f32 = jnp.float32
    srcp, dstp = _prep_edges(edge_index)

    deg16 = _deg_kernel(dstp)

    nslx = IN_DIM // SW
    nsl2 = OUT_DIM // SW
    dinv, xp = pl.pallas_call(
        _stage_ab_body,
        grid=(GRID,),
        in_specs=[pl.BlockSpec((NC, RB, 16), lambda i: (0, i, 0)),
                  pl.BlockSpec((RB, IN_DIM), lambda i: (i, 0))],
        out_specs=[pl.BlockSpec((RB, 1), lambda i: (i, 0)),
                   pl.BlockSpec((nslx, RB, SW), lambda i: (0, i, 0))],
        out_shape=[jax.ShapeDtypeStruct((N, 1), f32),
                   jax.ShapeDtypeStruct((nslx, N, SW), f32)],
    )(deg16, x)

    agg1 = _agg_kernel(srcp, dstp, xp)

    h2p = pl.pallas_call(
        _stage_c_body,
        grid=(GRID,),
        in_specs=[pl.BlockSpec((nslx, RB, SW), lambda i: (0, i, 0)),
                  pl.BlockSpec((nslx, RB, SW), lambda i: (0, i, 0)),
                  pl.BlockSpec((RB, 1), lambda i: (i, 0)),
                  pl.BlockSpec((1, HID_DIM), lambda i: (0, 0)),
                  pl.BlockSpec((IN_DIM, HID_DIM), lambda i: (0, 0)),
                  pl.BlockSpec((HID_DIM, OUT_DIM), lambda i: (0, 0))],
        out_specs=pl.BlockSpec((nsl2, RB, SW), lambda i: (0, i, 0)),
        out_shape=jax.ShapeDtypeStruct((nsl2, N, SW), f32),
    )(agg1, xp, dinv, b1.reshape(1, HID_DIM), W1, W2)

    agg2 = _agg_kernel(srcp, dstp, h2p)

    out = pl.pallas_call(
        _stage_d_body,
        grid=(GRID,),
        in_specs=[pl.BlockSpec((nsl2, RB, SW), lambda i: (0, i, 0)),
                  pl.BlockSpec((nsl2, RB, SW), lambda i: (0, i, 0)),
                  pl.BlockSpec((RB, 1), lambda i: (i, 0)),
                  pl.BlockSpec((1, OUT_DIM), lambda i: (0, 0))],
        out_specs=pl.BlockSpec((RB, OUT_DIM), lambda i: (i, 0)),
        out_shape=jax.ShapeDtypeStruct((N, OUT_DIM), f32),
    )(agg2, h2p, dinv, b2.reshape(1, OUT_DIM))
    return out
